# baseline (speedup 1.0000x reference)
import functools

import jax
import jax.numpy as jnp
from jax import lax
from jax.experimental import pallas as pl
from jax.experimental.pallas import tpu as pltpu
from jax.experimental.pallas import tpu_sc as plsc

_NUM_MEMORY = 100000
_NUM_DIMS = 64
_BATCH = 1024
_HIST = 50
_HIST_PAD = 64

_NC = 2
_NS = 16
_L = 16
_NW = _NC * _NS
_B_PER_W = _BATCH // _NW
_IDR = 128
_IDROWS = _BATCH * _HIST_PAD // _IDR
_IDR_PER_W = _IDROWS // _NW
_SLICE = 6272
_MASK_PAD = _NS * _SLICE


@functools.partial(
    pl.kernel,
    mesh=plsc.VectorSubcoreMesh(core_axis_name="c", subcore_axis_name="s"),
    compiler_params=pltpu.CompilerParams(needs_layout_passes=False),
    out_type=[
        jax.ShapeDtypeStruct((_BATCH, _NUM_DIMS), jnp.float32),
        jax.ShapeDtypeStruct((_MASK_PAD,), jnp.int32),
        jax.ShapeDtypeStruct((_MASK_PAD,), jnp.int32),
    ],
    scratch_types=[
        pltpu.VMEM((_B_PER_W,), jnp.int32),
        pltpu.VMEM((_IDR_PER_W, _IDR), jnp.int32),
        pltpu.VMEM((_IDR,), jnp.int32),
        pltpu.VMEM((_SLICE,), jnp.int32),
        pltpu.VMEM_SHARED((_MASK_PAD,), jnp.int32),
        pltpu.SemaphoreType.DMA,
        pltpu.SemaphoreType.DMA,
    ],
)
def _sc_kernel(src_hbm, tgt_hbm, mem_hbm, vecs_hbm, m0_hbm, m1_hbm,
               sidx_v, stage_v, ones_v, zbuf_v, shared, gsem, ssem):
    cid = lax.axis_index("c")
    sid = lax.axis_index("s")
    wid = sid * _NC + cid

    base = wid * _B_PER_W
    pltpu.sync_copy(src_hbm.at[pl.ds(base, _B_PER_W)], sidx_v)
    gather_copies = []
    for g in range(_B_PER_W // _L):
        v = sidx_v[pl.ds(g * _L, _L)]
        for j in range(_L):
            b = base + g * _L + j
            gather_copies.append(
                pltpu.async_copy(mem_hbm.at[v[j]], vecs_hbm.at[b], gsem)
            )

    ones = jnp.ones((_L,), jnp.int32)
    zeros = jnp.zeros((_L,), jnp.int32)
    for j in range(_IDR // _L):
        ones_v[pl.ds(j * _L, _L)] = ones

    def _zero_body(i, carry):
        zbuf_v[pl.ds(i * _L, _L)] = zeros
        return carry

    lax.fori_loop(0, _SLICE // _L, _zero_body, 0)

    pltpu.sync_copy(tgt_hbm.at[pl.ds(wid * _IDR_PER_W, _IDR_PER_W), :], stage_v)

    lo = sid * _SLICE
    pltpu.sync_copy(zbuf_v, shared.at[pl.ds(lo, _SLICE)])
    plsc.subcore_barrier()

    mask_copies = []
    for r in range(_IDR_PER_W):
        mask_copies.append(
            pltpu.async_copy(ones_v, shared.at[stage_v.at[r]], ssem)
        )
    for c in mask_copies:
        c.wait()
    plsc.subcore_barrier()

    @pl.when(cid == 0)
    def _():
        pltpu.sync_copy(shared.at[pl.ds(lo, _SLICE)], m0_hbm.at[pl.ds(lo, _SLICE)])

    @pl.when(cid == 1)
    def _():
        pltpu.sync_copy(shared.at[pl.ds(lo, _SLICE)], m1_hbm.at[pl.ds(lo, _SLICE)])

    for c in gather_copies:
        c.wait()


def kernel(src_ids, tgt_ids, memory):
    pad = (
        jnp.arange(_HIST_PAD - _HIST, dtype=jnp.int32)[None, :]
        + 16 * jnp.arange(_BATCH, dtype=jnp.int32)[:, None]
    ) % (_MASK_PAD - _NUM_MEMORY) + _NUM_MEMORY
    tgt_padded = jnp.concatenate([tgt_ids, pad], axis=1).reshape(_IDROWS, _IDR)
    vecs, m0, m1 = _sc_kernel(src_ids, tgt_padded, memory)
    connected_mask = (m0 | m1)[:_NUM_MEMORY].astype(jnp.bool_)
    return (vecs, connected_mask)


# speedup vs baseline: 1.0067x; 1.0067x over previous
import functools

import jax
import jax.numpy as jnp
from jax import lax
from jax.experimental import pallas as pl
from jax.experimental.pallas import tpu as pltpu
from jax.experimental.pallas import tpu_sc as plsc

_NUM_MEMORY = 100000
_NUM_DIMS = 64
_BATCH = 1024
_HIST = 50
_HIST_PAD = 64

_NC = 2
_NS = 16
_L = 16
_NW = _NC * _NS
_B_PER_W = _BATCH // _NW
_IDR = 128
_IDROWS = _BATCH * _HIST_PAD // _IDR
_IDR_PER_W = _IDROWS // _NW
_SLICE = 6272
_MASK_PAD = _NS * _SLICE


@functools.partial(
    pl.kernel,
    mesh=plsc.VectorSubcoreMesh(core_axis_name="c", subcore_axis_name="s"),
    out_type=[
        jax.ShapeDtypeStruct((_BATCH, _NUM_DIMS), jnp.float32),
        jax.ShapeDtypeStruct((_MASK_PAD,), jnp.int32),
        jax.ShapeDtypeStruct((_MASK_PAD,), jnp.int32),
    ],
    scratch_types=[
        pltpu.VMEM((_B_PER_W,), jnp.int32),
        pltpu.VMEM((_IDR_PER_W, _IDR), jnp.int32),
        pltpu.VMEM((_IDR,), jnp.int32),
        pltpu.VMEM((_SLICE,), jnp.int32),
        pltpu.VMEM_SHARED((_MASK_PAD,), jnp.int32),
        pltpu.SemaphoreType.DMA,
        pltpu.SemaphoreType.DMA,
    ],
)
def _sc_kernel(src_hbm, tgt_hbm, mem_hbm, vecs_hbm, m0_hbm, m1_hbm,
               sidx_v, stage_v, ones_v, zbuf_v, shared, gsem, ssem):
    cid = lax.axis_index("c")
    sid = lax.axis_index("s")
    wid = sid * _NC + cid

    base = wid * _B_PER_W
    pltpu.sync_copy(src_hbm.at[pl.ds(base, _B_PER_W)], sidx_v)
    gather_copies = []
    for g in range(_B_PER_W // _L):
        v = sidx_v[pl.ds(g * _L, _L)]
        for j in range(_L):
            b = base + g * _L + j
            gather_copies.append(
                pltpu.async_copy(mem_hbm.at[v[j]], vecs_hbm.at[b], gsem)
            )

    ones = jnp.ones((_L,), jnp.int32)
    zeros = jnp.zeros((_L,), jnp.int32)
    for j in range(_IDR // _L):
        ones_v[pl.ds(j * _L, _L)] = ones

    def _zero_body(i, carry):
        zbuf_v[pl.ds(i * _L, _L)] = zeros
        return carry

    lax.fori_loop(0, _SLICE // _L, _zero_body, 0)

    pltpu.sync_copy(tgt_hbm.at[pl.ds(wid * _IDR_PER_W, _IDR_PER_W), :], stage_v)

    lo = sid * _SLICE
    pltpu.sync_copy(zbuf_v, shared.at[pl.ds(lo, _SLICE)])
    plsc.subcore_barrier()

    mask_copies = []
    for r in range(_IDR_PER_W):
        mask_copies.append(
            pltpu.async_copy(ones_v, shared.at[stage_v.at[r]], ssem)
        )
    for c in mask_copies:
        c.wait()
    plsc.subcore_barrier()

    @pl.when(cid == 0)
    def _():
        pltpu.sync_copy(shared.at[pl.ds(lo, _SLICE)], m0_hbm.at[pl.ds(lo, _SLICE)])

    @pl.when(cid == 1)
    def _():
        pltpu.sync_copy(shared.at[pl.ds(lo, _SLICE)], m1_hbm.at[pl.ds(lo, _SLICE)])

    for c in gather_copies:
        c.wait()


def kernel(src_ids, tgt_ids, memory):
    pad = (
        jnp.arange(_HIST_PAD - _HIST, dtype=jnp.int32)[None, :]
        + 16 * jnp.arange(_BATCH, dtype=jnp.int32)[:, None]
    ) % (_MASK_PAD - _NUM_MEMORY) + _NUM_MEMORY
    tgt_padded = jnp.concatenate([tgt_ids, pad], axis=1).reshape(_IDROWS, _IDR)
    vecs, m0, m1 = _sc_kernel(src_ids, tgt_padded, memory)
    connected_mask = (m0 | m1)[:_NUM_MEMORY].astype(jnp.bool_)
    return (vecs, connected_mask)


# speedup vs baseline: 2.2110x; 2.1963x over previous
import functools

import jax
import jax.numpy as jnp
from jax import lax
from jax.experimental import pallas as pl
from jax.experimental.pallas import tpu as pltpu
from jax.experimental.pallas import tpu_sc as plsc

_NUM_MEMORY = 100000
_NUM_DIMS = 64
_BATCH = 1024
_HIST = 50
_HIST_PAD = 64

_NC = 2
_NS = 16
_L = 16
_NW = _NC * _NS
_D_PER_W = _NUM_DIMS // _NW
_IDR = 128
_IDROWS = _BATCH * _HIST_PAD // _IDR
_IDR_PER_W = _IDROWS // _NW
_SLICE = 6272
_MASK_PAD = _NS * _SLICE


@functools.partial(
    pl.kernel,
    mesh=plsc.VectorSubcoreMesh(core_axis_name="c", subcore_axis_name="s"),
    compiler_params=pltpu.CompilerParams(needs_layout_passes=False),
    out_type=[
        jax.ShapeDtypeStruct((_NUM_DIMS, _BATCH), jnp.float32),
        jax.ShapeDtypeStruct((_MASK_PAD,), jnp.int32),
        jax.ShapeDtypeStruct((_MASK_PAD,), jnp.int32),
    ],
    scratch_types=[
        pltpu.VMEM((_BATCH,), jnp.int32),
        pltpu.VMEM((_NUM_MEMORY,), jnp.float32),
        pltpu.VMEM((_BATCH,), jnp.float32),
        pltpu.VMEM((_IDR_PER_W, _IDR), jnp.int32),
        pltpu.VMEM((_IDR,), jnp.int32),
        pltpu.VMEM((_SLICE,), jnp.int32),
        pltpu.VMEM_SHARED((_MASK_PAD,), jnp.int32),
        pltpu.SemaphoreType.DMA,
        pltpu.SemaphoreType.DMA,
        pltpu.SemaphoreType.DMA,
        pltpu.SemaphoreType.DMA,
    ],
)
def _sc_kernel(src_hbm, tgt_hbm, memt_hbm, vecst_hbm, m0_hbm, m1_hbm,
               sidx_v, slab_v, orow_v, stage_v, ones_v, zbuf_v, shared,
               gsem, ssem, osem, isem):
    cid = lax.axis_index("c")
    sid = lax.axis_index("s")
    wid = sid * _NC + cid
    d0 = wid * _D_PER_W

    sidx_copy = pltpu.async_copy(src_hbm, sidx_v, isem)
    slab_copy = pltpu.async_copy(memt_hbm.at[d0], slab_v, gsem)

    ones = jnp.ones((_L,), jnp.int32)
    zeros = jnp.zeros((_L,), jnp.int32)
    for j in range(_IDR // _L):
        ones_v[pl.ds(j * _L, _L)] = ones

    def _zero_body(i, carry):
        zbuf_v[pl.ds(i * _L, _L)] = zeros
        return carry

    lax.fori_loop(0, _SLICE // _L, _zero_body, 0)

    pltpu.sync_copy(tgt_hbm.at[pl.ds(wid * _IDR_PER_W, _IDR_PER_W), :], stage_v)

    lo = sid * _SLICE
    pltpu.sync_copy(zbuf_v, shared.at[pl.ds(lo, _SLICE)])
    plsc.subcore_barrier()

    mask_copies = []
    for r in range(_IDR_PER_W):
        mask_copies.append(
            pltpu.async_copy(ones_v, shared.at[stage_v.at[r]], ssem)
        )
    for c in mask_copies:
        c.wait()
    plsc.subcore_barrier()

    @pl.when(cid == 0)
    def _():
        pltpu.sync_copy(shared.at[pl.ds(lo, _SLICE)], m0_hbm.at[pl.ds(lo, _SLICE)])

    @pl.when(cid == 1)
    def _():
        pltpu.sync_copy(shared.at[pl.ds(lo, _SLICE)], m1_hbm.at[pl.ds(lo, _SLICE)])

    sidx_copy.wait()
    out_copies = []
    for k in range(_D_PER_W):
        slab_copy.wait()

        def _gather_body(i, carry):
            idx = sidx_v[pl.ds(i * _L, _L)]
            orow_v[pl.ds(i * _L, _L)] = plsc.load_gather(slab_v, [idx])
            return carry

        lax.fori_loop(0, _BATCH // _L, _gather_body, 0)
        out_copies.append(
            pltpu.async_copy(orow_v, vecst_hbm.at[d0 + k], osem)
        )
        if k + 1 < _D_PER_W:
            out_copies.pop(0).wait()
            slab_copy = pltpu.async_copy(memt_hbm.at[d0 + k + 1], slab_v, gsem)
    for c in out_copies:
        c.wait()


def kernel(src_ids, tgt_ids, memory):
    pad = (
        jnp.arange(_HIST_PAD - _HIST, dtype=jnp.int32)[None, :]
        + 16 * jnp.arange(_BATCH, dtype=jnp.int32)[:, None]
    ) % (_MASK_PAD - _NUM_MEMORY) + _NUM_MEMORY
    tgt_padded = jnp.concatenate([tgt_ids, pad], axis=1).reshape(_IDROWS, _IDR)
    vecs_t, m0, m1 = _sc_kernel(src_ids, tgt_padded, memory.T)
    connected_mask = (m0 | m1)[:_NUM_MEMORY].astype(jnp.bool_)
    return (vecs_t.T, connected_mask)
